# baseline (speedup 1.0000x reference)
import functools

import jax
import jax.numpy as jnp
from jax import lax
from jax.experimental import pallas as pl
from jax.experimental.pallas import tpu as pltpu
from jax.experimental.pallas import tpu_sc as plsc

NC = 2
NS = 16
NW = NC * NS
CH = 64
NBUF = 4
LAG = 2
KW = 8
DWIN = 16
DW = 128

_MESH = plsc.VectorSubcoreMesh(core_axis_name="c", subcore_axis_name="s")


def _deg_sc(dst3, ones16, zer16, n_pad, nch):
    rpt = n_pad // NS
    out_sds = jax.ShapeDtypeStruct((NC, n_pad, DW), jnp.float32)

    @functools.partial(
        pl.kernel,
        out_type=out_sds,
        mesh=_MESH,
        scratch_types=[
            pltpu.VMEM((nch, CH), jnp.int32),
            pltpu.VMEM((CH, DW), jnp.float32),
            pltpu.VMEM_SHARED((n_pad, DW), jnp.float32),
            pltpu.SemaphoreType.DMA,
        ],
    )
    def deg(dst_hbm, ones_hbm, zer_hbm, p_hbm, dst_v, ones_v, acc_sh, dsem):
        c = lax.axis_index("c")
        s = lax.axis_index("s")
        w = s * NC + c
        base = s * rpt
        pltpu.sync_copy(zer_hbm, acc_sh.at[pl.ds(base, rpt)])
        pltpu.sync_copy(ones_hbm, ones_v)
        pltpu.sync_copy(dst_hbm.at[w], dst_v)
        plsc.subcore_barrier()

        @pl.loop(0, nch)
        def _(j):
            pltpu.async_copy(ones_v, acc_sh.at[dst_v.at[j]], dsem, add=True)

            @pl.when(j >= DWIN)
            def _():
                pltpu.make_async_copy(ones_v, acc_sh.at[dst_v.at[j - DWIN]],
                                      dsem).wait()

        @pl.loop(nch - DWIN, nch)
        def _(j):
            pltpu.make_async_copy(ones_v, acc_sh.at[dst_v.at[j]], dsem).wait()

        plsc.subcore_barrier()
        pltpu.sync_copy(acc_sh.at[pl.ds(base, rpt)],
                        p_hbm.at[c].at[pl.ds(base, rpt)])

    return deg(dst3, ones16, zer16)


def _agg_sc(g, src3, dst3, zerF, n_pad, nch):
    F = g.shape[1]
    rpt = n_pad // NS
    out_sds = jax.ShapeDtypeStruct((NC, n_pad, F), jnp.float32)

    ng = nch // KW

    @functools.partial(
        pl.kernel,
        out_type=out_sds,
        mesh=_MESH,
        scratch_types=(
            [pltpu.VMEM((2, KW * CH), jnp.int32),
             pltpu.VMEM((2, KW, CH), jnp.int32)]
            + [pltpu.VMEM((CH, F), jnp.float32) for _ in range(NBUF)]
            + [pltpu.VMEM_SHARED((n_pad, F), jnp.float32)]
            + [pltpu.SemaphoreType.DMA for _ in range(NBUF + 1)]
        ),
    )
    def agg(g_hbm, src_hbm, dst_hbm, zer_hbm, p_hbm, sw, dw, *rest):
        rb = rest[:NBUF]
        acc_sh = rest[NBUF]
        sem = rest[NBUF + 1:NBUF + 1 + NBUF]
        wsem = rest[NBUF + 1 + NBUF]
        c = lax.axis_index("c")
        s = lax.axis_index("s")
        w = s * NC + c
        base = s * rpt
        pltpu.sync_copy(zer_hbm, acc_sh.at[pl.ds(base, rpt)])
        pltpu.sync_copy(src_hbm.at[w].at[pl.ds(0, KW * CH)], sw.at[0])
        pltpu.sync_copy(dst_hbm.at[w].at[pl.ds(0, KW)], dw.at[0])
        plsc.subcore_barrier()

        for b in range(LAG):
            pltpu.async_copy(g_hbm.at[sw.at[0].at[pl.ds(b * CH, CH)]],
                             rb[b], sem[b])

        @pl.loop(0, ng)
        def _(h):
            p = h % 2
            q = (h + 1) % 2

            def swin(par, i):
                return sw.at[par].at[pl.ds(i * CH, CH)]

            for i in range(KW):
                jb = h * KW + i
                b = i % NBUF
                pltpu.make_async_copy(g_hbm.at[swin(p, i)], rb[b],
                                      sem[b]).wait()
                pltpu.async_copy(rb[b], acc_sh.at[dw.at[p].at[i]], sem[b],
                                 add=True)
                bn = (b + LAG) % NBUF

                @pl.when(jb >= NBUF - LAG)
                def _():
                    pltpu.make_async_copy(rb[bn], acc_sh.at[dw.at[p].at[i]],
                                          sem[bn]).wait()

                @pl.when(jb + LAG < nch)
                def _():
                    if i + LAG < KW:
                        pltpu.async_copy(g_hbm.at[swin(p, i + LAG)], rb[bn],
                                         sem[bn])
                    else:
                        pltpu.async_copy(g_hbm.at[swin(q, i + LAG - KW)],
                                         rb[bn], sem[bn])

                if i == 2:
                    @pl.when(h + 1 < ng)
                    def _():
                        pltpu.async_copy(
                            src_hbm.at[w].at[pl.ds((h + 1) * KW * CH,
                                                   KW * CH)],
                            sw.at[q], wsem)
                        pltpu.async_copy(
                            dst_hbm.at[w].at[pl.ds((h + 1) * KW, KW)],
                            dw.at[q], wsem)

                if i == KW - LAG:
                    @pl.when(h + 1 < ng)
                    def _():
                        pltpu.make_async_copy(
                            src_hbm.at[w].at[pl.ds((h + 1) * KW * CH,
                                                   KW * CH)],
                            sw.at[q], wsem).wait()
                        pltpu.make_async_copy(
                            dst_hbm.at[w].at[pl.ds((h + 1) * KW, KW)],
                            dw.at[q], wsem).wait()

        for t in range(nch - (NBUF - LAG), nch):
            pltpu.make_async_copy(rb[t % NBUF], acc_sh.at[dw.at[0].at[0]],
                                  sem[t % NBUF]).wait()
        plsc.subcore_barrier()
        pltpu.sync_copy(acc_sh.at[pl.ds(base, rpt)],
                        p_hbm.at[c].at[pl.ds(base, rpt)])

    return agg(g, src3, dst3, zerF)


_ROWS = 1000


def _mm_tc(x, W):
    N, F = x.shape

    def body(x_ref, w_ref, o_ref):
        o_ref[...] = jnp.dot(x_ref[...], w_ref[...],
                             preferred_element_type=jnp.float32)

    return pl.pallas_call(
        body,
        grid=(N // _ROWS,),
        in_specs=[pl.BlockSpec((_ROWS, F), lambda i: (i, 0)),
                  pl.BlockSpec((F, F), lambda i: (0, 0))],
        out_specs=pl.BlockSpec((_ROWS, F), lambda i: (i, 0)),
        out_shape=jax.ShapeDtypeStruct((N, F), jnp.float32),
    )(x, W)


def _scale_tc(h, d0, d1):
    N, F = h.shape

    def body(h_ref, d0_ref, d1_ref, o_ref):
        dinv = lax.rsqrt(d0_ref[:, 0:1] + d1_ref[:, 0:1] + 1.0)
        o_ref[...] = h_ref[...] * dinv

    return pl.pallas_call(
        body,
        grid=(N // _ROWS,),
        in_specs=[pl.BlockSpec((_ROWS, F), lambda i: (i, 0)),
                  pl.BlockSpec((_ROWS, DW), lambda i: (i, 0)),
                  pl.BlockSpec((_ROWS, DW), lambda i: (i, 0))],
        out_specs=pl.BlockSpec((_ROWS, F), lambda i: (i, 0)),
        out_shape=jax.ShapeDtypeStruct((N, F), jnp.float32),
    )(h, d0, d1)


def _epi_mm_tc(a0, a1, g, d0, d1, b, W):
    N, F = g.shape

    def body(a0_ref, a1_ref, g_ref, d0_ref, d1_ref, b_ref, w_ref, o_ref):
        dinv = lax.rsqrt(d0_ref[:, 0:1] + d1_ref[:, 0:1] + 1.0)
        h = jax.nn.sigmoid(
            dinv * (a0_ref[...] + a1_ref[...] + g_ref[...]) + b_ref[...])
        o_ref[...] = jnp.dot(h, w_ref[...],
                             preferred_element_type=jnp.float32) * dinv

    return pl.pallas_call(
        body,
        grid=(N // _ROWS,),
        in_specs=[pl.BlockSpec((_ROWS, F), lambda i: (i, 0)),
                  pl.BlockSpec((_ROWS, F), lambda i: (i, 0)),
                  pl.BlockSpec((_ROWS, F), lambda i: (i, 0)),
                  pl.BlockSpec((_ROWS, DW), lambda i: (i, 0)),
                  pl.BlockSpec((_ROWS, DW), lambda i: (i, 0)),
                  pl.BlockSpec((1, F), lambda i: (0, 0)),
                  pl.BlockSpec((F, F), lambda i: (0, 0))],
        out_specs=pl.BlockSpec((_ROWS, F), lambda i: (i, 0)),
        out_shape=jax.ShapeDtypeStruct((N, F), jnp.float32),
    )(a0, a1, g, d0, d1, b, W)


def _epi_tc(a0, a1, g, d0, d1, b):
    N, F = g.shape

    def body(a0_ref, a1_ref, g_ref, d0_ref, d1_ref, b_ref, o_ref):
        dinv = lax.rsqrt(d0_ref[:, 0:1] + d1_ref[:, 0:1] + 1.0)
        o_ref[...] = jax.nn.sigmoid(
            dinv * (a0_ref[...] + a1_ref[...] + g_ref[...]) + b_ref[...])

    return pl.pallas_call(
        body,
        grid=(N // _ROWS,),
        in_specs=[pl.BlockSpec((_ROWS, F), lambda i: (i, 0)),
                  pl.BlockSpec((_ROWS, F), lambda i: (i, 0)),
                  pl.BlockSpec((_ROWS, F), lambda i: (i, 0)),
                  pl.BlockSpec((_ROWS, DW), lambda i: (i, 0)),
                  pl.BlockSpec((_ROWS, DW), lambda i: (i, 0)),
                  pl.BlockSpec((1, F), lambda i: (0, 0))],
        out_specs=pl.BlockSpec((_ROWS, F), lambda i: (i, 0)),
        out_shape=jax.ShapeDtypeStruct((N, F), jnp.float32),
    )(a0, a1, g, d0, d1, b)


def kernel(x, edge_index, W1, b1, W2, b2):
    N, F = x.shape
    E = edge_index.shape[1]
    nch = -(-E // (NW * CH))
    nch = KW * (-(-nch // KW))
    e_pad = NW * nch * CH
    n_pad = NS * 8 * (-(-(N + 1) // (NS * 8)))

    src3 = jnp.concatenate(
        [edge_index[0], jnp.zeros((e_pad - E,), jnp.int32)]).reshape(NW, nch * CH)
    dst3 = jnp.concatenate(
        [edge_index[1], jnp.full((e_pad - E,), N, jnp.int32)]).reshape(NW, nch, CH)
    ones16 = jnp.ones((CH, DW), jnp.float32)
    zer16 = jnp.zeros((n_pad // NS, DW), jnp.float32)
    zerF = jnp.zeros((n_pad // NS, F), jnp.float32)
    b1r = b1.reshape(1, F)
    b2r = b2.reshape(1, F)

    dp = _deg_sc(dst3, ones16, zer16, n_pad, nch)
    d0, d1 = dp[0, :N], dp[1, :N]
    h1 = _mm_tc(x, W1)
    g1 = _scale_tc(h1, d0, d1)
    a = _agg_sc(g1, src3, dst3, zerF, n_pad, nch)
    g2 = _epi_mm_tc(a[0, :N], a[1, :N], g1, d0, d1, b1r, W2)
    c = _agg_sc(g2, src3, dst3, zerF, n_pad, nch)
    return _epi_tc(c[0, :N], c[1, :N], g2, d0, d1, b2r)


# speedup vs baseline: 1.3399x; 1.3399x over previous
import functools

import jax
import jax.numpy as jnp
from jax import lax
from jax.experimental import pallas as pl
from jax.experimental.pallas import tpu as pltpu
from jax.experimental.pallas import tpu_sc as plsc

NC = 2
NS = 16
NW = NC * NS
CH = 128
DWIN = 16
DW = 128

_MESH = plsc.VectorSubcoreMesh(core_axis_name="c", subcore_axis_name="s")


def _deg_sc(dst3, ones16, zer16, n_pad, nch):
    rpt = n_pad // NS
    out_sds = jax.ShapeDtypeStruct((NC, n_pad, DW), jnp.float32)

    @functools.partial(
        pl.kernel,
        out_type=out_sds,
        mesh=_MESH,
        scratch_types=[
            pltpu.VMEM((nch, CH), jnp.int32),
            pltpu.VMEM((CH, DW), jnp.float32),
            pltpu.VMEM_SHARED((n_pad, DW), jnp.float32),
            pltpu.SemaphoreType.DMA,
        ],
    )
    def deg(dst_hbm, ones_hbm, zer_hbm, p_hbm, dst_v, ones_v, acc_sh, dsem):
        c = lax.axis_index("c")
        s = lax.axis_index("s")
        w = s * NC + c
        base = s * rpt
        pltpu.sync_copy(zer_hbm, acc_sh.at[pl.ds(base, rpt)])
        pltpu.sync_copy(ones_hbm, ones_v)
        pltpu.sync_copy(dst_hbm.at[w], dst_v)
        plsc.subcore_barrier()

        @pl.loop(0, nch)
        def _(j):
            pltpu.async_copy(ones_v, acc_sh.at[dst_v.at[j]], dsem, add=True)

            @pl.when(j >= DWIN)
            def _():
                pltpu.make_async_copy(ones_v, acc_sh.at[dst_v.at[j - DWIN]],
                                      dsem).wait()

        @pl.loop(nch - DWIN, nch)
        def _(j):
            pltpu.make_async_copy(ones_v, acc_sh.at[dst_v.at[j]], dsem).wait()

        plsc.subcore_barrier()
        pltpu.sync_copy(acc_sh.at[pl.ds(base, rpt)],
                        p_hbm.at[c].at[pl.ds(base, rpt)])

    return deg(dst3, ones16, zer16)


def _agg_sc(g, src3, dst3, zerF, n_pad, nch):
    F = g.shape[1]
    rpt = n_pad // NS
    out_sds = jax.ShapeDtypeStruct((NC, n_pad, F), jnp.float32)

    @functools.partial(
        pl.kernel,
        out_type=out_sds,
        mesh=_MESH,
        scratch_types=[
            pltpu.VMEM((nch, CH), jnp.int32),
            pltpu.VMEM((nch, CH), jnp.int32),
            pltpu.VMEM((CH, F), jnp.float32),
            pltpu.VMEM_SHARED((n_pad, F), jnp.float32),
            pltpu.SemaphoreType.DMA,
        ],
    )
    def agg(g_hbm, src_hbm, dst_hbm, zer_hbm, p_hbm, sv, dv, rb, acc_sh, sem):
        c = lax.axis_index("c")
        s = lax.axis_index("s")
        w = s * NC + c
        base = s * rpt
        pltpu.sync_copy(zer_hbm, acc_sh.at[pl.ds(base, rpt)])
        pltpu.sync_copy(src_hbm.at[w], sv)
        pltpu.sync_copy(dst_hbm.at[w], dv)
        plsc.subcore_barrier()

        @pl.loop(0, nch)
        def _(j):
            pltpu.sync_copy(g_hbm.at[sv.at[j]], rb)
            pltpu.async_copy(rb, acc_sh.at[dv.at[j]], sem, add=True)
            pltpu.make_async_copy(rb, acc_sh.at[dv.at[j]], sem).wait()

        plsc.subcore_barrier()
        pltpu.sync_copy(acc_sh.at[pl.ds(base, rpt)],
                        p_hbm.at[c].at[pl.ds(base, rpt)])

    return agg(g, src3, dst3, zerF)


_ROWS = 1000


def _mm_tc(x, W):
    N, F = x.shape

    def body(x_ref, w_ref, o_ref):
        o_ref[...] = jnp.dot(x_ref[...], w_ref[...],
                             preferred_element_type=jnp.float32)

    return pl.pallas_call(
        body,
        grid=(N // _ROWS,),
        in_specs=[pl.BlockSpec((_ROWS, F), lambda i: (i, 0)),
                  pl.BlockSpec((F, F), lambda i: (0, 0))],
        out_specs=pl.BlockSpec((_ROWS, F), lambda i: (i, 0)),
        out_shape=jax.ShapeDtypeStruct((N, F), jnp.float32),
    )(x, W)


def _scale_tc(h, d0, d1):
    N, F = h.shape

    def body(h_ref, d0_ref, d1_ref, o_ref):
        dinv = lax.rsqrt(d0_ref[:, 0:1] + d1_ref[:, 0:1] + 1.0)
        o_ref[...] = h_ref[...] * dinv

    return pl.pallas_call(
        body,
        grid=(N // _ROWS,),
        in_specs=[pl.BlockSpec((_ROWS, F), lambda i: (i, 0)),
                  pl.BlockSpec((_ROWS, DW), lambda i: (i, 0)),
                  pl.BlockSpec((_ROWS, DW), lambda i: (i, 0))],
        out_specs=pl.BlockSpec((_ROWS, F), lambda i: (i, 0)),
        out_shape=jax.ShapeDtypeStruct((N, F), jnp.float32),
    )(h, d0, d1)


def _epi_mm_tc(a0, a1, g, d0, d1, b, W):
    N, F = g.shape

    def body(a0_ref, a1_ref, g_ref, d0_ref, d1_ref, b_ref, w_ref, o_ref):
        dinv = lax.rsqrt(d0_ref[:, 0:1] + d1_ref[:, 0:1] + 1.0)
        h = jax.nn.sigmoid(
            dinv * (a0_ref[...] + a1_ref[...] + g_ref[...]) + b_ref[...])
        o_ref[...] = jnp.dot(h, w_ref[...],
                             preferred_element_type=jnp.float32) * dinv

    return pl.pallas_call(
        body,
        grid=(N // _ROWS,),
        in_specs=[pl.BlockSpec((_ROWS, F), lambda i: (i, 0)),
                  pl.BlockSpec((_ROWS, F), lambda i: (i, 0)),
                  pl.BlockSpec((_ROWS, F), lambda i: (i, 0)),
                  pl.BlockSpec((_ROWS, DW), lambda i: (i, 0)),
                  pl.BlockSpec((_ROWS, DW), lambda i: (i, 0)),
                  pl.BlockSpec((1, F), lambda i: (0, 0)),
                  pl.BlockSpec((F, F), lambda i: (0, 0))],
        out_specs=pl.BlockSpec((_ROWS, F), lambda i: (i, 0)),
        out_shape=jax.ShapeDtypeStruct((N, F), jnp.float32),
    )(a0, a1, g, d0, d1, b, W)


def _epi_tc(a0, a1, g, d0, d1, b):
    N, F = g.shape

    def body(a0_ref, a1_ref, g_ref, d0_ref, d1_ref, b_ref, o_ref):
        dinv = lax.rsqrt(d0_ref[:, 0:1] + d1_ref[:, 0:1] + 1.0)
        o_ref[...] = jax.nn.sigmoid(
            dinv * (a0_ref[...] + a1_ref[...] + g_ref[...]) + b_ref[...])

    return pl.pallas_call(
        body,
        grid=(N // _ROWS,),
        in_specs=[pl.BlockSpec((_ROWS, F), lambda i: (i, 0)),
                  pl.BlockSpec((_ROWS, F), lambda i: (i, 0)),
                  pl.BlockSpec((_ROWS, F), lambda i: (i, 0)),
                  pl.BlockSpec((_ROWS, DW), lambda i: (i, 0)),
                  pl.BlockSpec((_ROWS, DW), lambda i: (i, 0)),
                  pl.BlockSpec((1, F), lambda i: (0, 0))],
        out_specs=pl.BlockSpec((_ROWS, F), lambda i: (i, 0)),
        out_shape=jax.ShapeDtypeStruct((N, F), jnp.float32),
    )(a0, a1, g, d0, d1, b)


def kernel(x, edge_index, W1, b1, W2, b2):
    N, F = x.shape
    E = edge_index.shape[1]
    nch = -(-E // (NW * CH))
    e_pad = NW * nch * CH
    n_pad = NS * 8 * (-(-(N + 1) // (NS * 8)))

    src3 = jnp.concatenate(
        [edge_index[0], jnp.zeros((e_pad - E,), jnp.int32)]).reshape(NW, nch, CH)
    dst3 = jnp.concatenate(
        [edge_index[1], jnp.full((e_pad - E,), N, jnp.int32)]).reshape(NW, nch, CH)
    ones16 = jnp.ones((CH, DW), jnp.float32)
    zer16 = jnp.zeros((n_pad // NS, DW), jnp.float32)
    zerF = jnp.zeros((n_pad // NS, F), jnp.float32)
    b1r = b1.reshape(1, F)
    b2r = b2.reshape(1, F)

    dp = _deg_sc(dst3, ones16, zer16, n_pad, nch)
    d0, d1 = dp[0, :N], dp[1, :N]
    h1 = _mm_tc(x, W1)
    g1 = _scale_tc(h1, d0, d1)
    a = _agg_sc(g1, src3, dst3, zerF, n_pad, nch)
    g2 = _epi_mm_tc(a[0, :N], a[1, :N], g1, d0, d1, b1r, W2)
    c = _agg_sc(g2, src3, dst3, zerF, n_pad, nch)
    return _epi_tc(c[0, :N], c[1, :N], g2, d0, d1, b2r)
